# baseline (speedup 1.0000x reference)
import jax
import jax.numpy as jnp
from jax import lax
from jax.experimental import pallas as pl
from jax.experimental.pallas import tpu as pltpu

BATCH = 4096
FEAT = 128
NROWS = 100000
BLOCK_N = 256
NBUF = 8
SPLIT = 2
COLS_PER = BATCH // SPLIT


def _make_body(nblocks, tail):
    last = nblocks - 1

    def _copy(o_hbm, scratch, sems, s, block, nrows, k):
        band = nrows // SPLIT
        return pltpu.make_async_copy(
            scratch.at[s, pl.ds(k * band, band), :],
            o_hbm.at[pl.ds(block * BLOCK_N + k * band, band), :],
            sems.at[s, k],
        )

    def _body(params_ref, x_ref, m_ref, o_hbm, scratch, xbf, sems):
        i = pl.program_id(0)
        s = lax.rem(i, NBUF)
        inv_t = 1.0 / params_ref[0]

        @pl.when(i == 0)
        def _():
            xbf[...] = x_ref[...].astype(jnp.bfloat16)

        @pl.when(i >= NBUF)
        def _():
            for k in range(SPLIT):
                _copy(o_hbm, scratch, sems, s, i - NBUF, BLOCK_N, k).wait()

        acc = lax.dot_general(
            m_ref[...].astype(jnp.bfloat16), xbf[...],
            dimension_numbers=(((1,), (1,)), ((), ())),
            preferred_element_type=jnp.float32,
        )
        scratch[s] = acc * inv_t

        @pl.when(i < last)
        def _():
            for k in range(SPLIT):
                _copy(o_hbm, scratch, sems, s, i, BLOCK_N, k).start()

        @pl.when(i == last)
        def _():
            s_last = last % NBUF
            for k in range(SPLIT):
                _copy(o_hbm, scratch, sems, s_last, last, tail, k).start()
            for k in range(SPLIT):
                _copy(o_hbm, scratch, sems, s_last, last, tail, k).wait()
            for d in range(1, min(NBUF, nblocks)):
                for k in range(SPLIT):
                    _copy(o_hbm, scratch, sems, (last - d) % NBUF, last - d,
                          BLOCK_N, k).wait()

    return _body


def kernel(x, y, memory, params):
    del y
    nblocks = pl.cdiv(NROWS, BLOCK_N)
    tail = NROWS - (nblocks - 1) * BLOCK_N
    out_t = pl.pallas_call(
        _make_body(nblocks, tail),
        grid=(nblocks,),
        in_specs=[
            pl.BlockSpec(memory_space=pltpu.SMEM),
            pl.BlockSpec((BATCH, FEAT), lambda j: (0, 0)),
            pl.BlockSpec((BLOCK_N, FEAT), lambda j: (j, 0)),
        ],
        out_specs=pl.BlockSpec(memory_space=pl.ANY),
        out_shape=jax.ShapeDtypeStruct((NROWS, BATCH), jnp.float32),
        scratch_shapes=[
            pltpu.VMEM((NBUF, BLOCK_N, BATCH), jnp.float32),
            pltpu.VMEM((BATCH, FEAT), jnp.bfloat16),
            pltpu.SemaphoreType.DMA((NBUF, SPLIT)),
        ],
        compiler_params=pltpu.CompilerParams(
            dimension_semantics=("arbitrary",),
        ),
    )(params, x, memory)
    return out_t.T


# speedup vs baseline: 1.0344x; 1.0344x over previous
import jax
import jax.numpy as jnp
from jax import lax
from jax.experimental import pallas as pl
from jax.experimental.pallas import tpu as pltpu

BATCH = 4096
FEAT = 128
NROWS = 100000
BLOCK_N = 640
NBUF = 4
SPLIT = 5
COLS_PER = BATCH // SPLIT


def _make_body(nblocks, tail):
    last = nblocks - 1

    def _copy(o_hbm, scratch, sems, s, block, nrows, k):
        band = nrows // SPLIT
        return pltpu.make_async_copy(
            scratch.at[s, pl.ds(k * band, band), :],
            o_hbm.at[pl.ds(block * BLOCK_N + k * band, band), :],
            sems.at[s, k],
        )

    def _body(params_ref, x_ref, m_ref, o_hbm, scratch, xbf, sems):
        i = pl.program_id(0)
        s = lax.rem(i, NBUF)
        inv_t = 1.0 / params_ref[0]

        @pl.when(i == 0)
        def _():
            xbf[...] = x_ref[...].astype(jnp.bfloat16)

        @pl.when(i >= NBUF)
        def _():
            for k in range(SPLIT):
                _copy(o_hbm, scratch, sems, s, i - NBUF, BLOCK_N, k).wait()

        acc = lax.dot_general(
            m_ref[...].astype(jnp.bfloat16), xbf[...],
            dimension_numbers=(((1,), (1,)), ((), ())),
            preferred_element_type=jnp.float32,
        )
        scratch[s] = acc * inv_t

        @pl.when(i < last)
        def _():
            for k in range(SPLIT):
                _copy(o_hbm, scratch, sems, s, i, BLOCK_N, k).start()

        @pl.when(i == last)
        def _():
            s_last = last % NBUF
            for k in range(SPLIT):
                _copy(o_hbm, scratch, sems, s_last, last, tail, k).start()
            for k in range(SPLIT):
                _copy(o_hbm, scratch, sems, s_last, last, tail, k).wait()
            for d in range(1, min(NBUF, nblocks)):
                for k in range(SPLIT):
                    _copy(o_hbm, scratch, sems, (last - d) % NBUF, last - d,
                          BLOCK_N, k).wait()

    return _body


def kernel(x, y, memory, params):
    del y
    nblocks = pl.cdiv(NROWS, BLOCK_N)
    tail = NROWS - (nblocks - 1) * BLOCK_N
    out_t = pl.pallas_call(
        _make_body(nblocks, tail),
        grid=(nblocks,),
        in_specs=[
            pl.BlockSpec(memory_space=pltpu.SMEM),
            pl.BlockSpec((BATCH, FEAT), lambda j: (0, 0)),
            pl.BlockSpec((BLOCK_N, FEAT), lambda j: (j, 0)),
        ],
        out_specs=pl.BlockSpec(memory_space=pl.ANY),
        out_shape=jax.ShapeDtypeStruct((NROWS, BATCH), jnp.float32),
        scratch_shapes=[
            pltpu.VMEM((NBUF, BLOCK_N, BATCH), jnp.float32),
            pltpu.VMEM((BATCH, FEAT), jnp.bfloat16),
            pltpu.SemaphoreType.DMA((NBUF, SPLIT)),
        ],
        compiler_params=pltpu.CompilerParams(
            dimension_semantics=("arbitrary",),
        ),
    )(params, x, memory)
    return out_t.T


# speedup vs baseline: 1.0386x; 1.0040x over previous
import jax
import jax.numpy as jnp
from jax import lax
from jax.experimental import pallas as pl
from jax.experimental.pallas import tpu as pltpu

BATCH = 4096
FEAT = 128
NROWS = 100000
BLOCK_N = 768
NBUF = 4
SPLIT = 4
COLS_PER = BATCH // SPLIT


def _make_body(nblocks, tail):
    last = nblocks - 1

    def _copy(o_hbm, scratch, sems, s, block, nrows, k):
        band = nrows // SPLIT
        return pltpu.make_async_copy(
            scratch.at[s, pl.ds(k * band, band), :],
            o_hbm.at[pl.ds(block * BLOCK_N + k * band, band), :],
            sems.at[s, k],
        )

    def _body(params_ref, x_ref, m_ref, o_hbm, scratch, xbf, sems):
        i = pl.program_id(0)
        s = lax.rem(i, NBUF)
        inv_t = 1.0 / params_ref[0]

        @pl.when(i == 0)
        def _():
            xbf[...] = x_ref[...].astype(jnp.bfloat16)

        @pl.when(i >= NBUF)
        def _():
            for k in range(SPLIT):
                _copy(o_hbm, scratch, sems, s, i - NBUF, BLOCK_N, k).wait()

        half = BLOCK_N // 2

        @pl.when(i < last)
        def _():
            for h in range(2):
                acc = lax.dot_general(
                    m_ref[pl.ds(h * half, half), :].astype(jnp.bfloat16),
                    xbf[...],
                    dimension_numbers=(((1,), (1,)), ((), ())),
                    preferred_element_type=jnp.float32,
                )
                scratch[s, pl.ds(h * half, half), :] = acc * inv_t
                for k in range(h * (SPLIT // 2), (h + 1) * (SPLIT // 2)):
                    _copy(o_hbm, scratch, sems, s, i, BLOCK_N, k).start()

        @pl.when(i == last)
        def _():
            s_last = last % NBUF
            acc = lax.dot_general(
                m_ref[pl.ds(0, tail), :].astype(jnp.bfloat16), xbf[...],
                dimension_numbers=(((1,), (1,)), ((), ())),
                preferred_element_type=jnp.float32,
            )
            scratch[s_last, pl.ds(0, tail), :] = acc * inv_t
            for k in range(SPLIT):
                _copy(o_hbm, scratch, sems, s_last, last, tail, k).start()
            for k in range(SPLIT):
                _copy(o_hbm, scratch, sems, s_last, last, tail, k).wait()
            for d in range(1, min(NBUF, nblocks)):
                for k in range(SPLIT):
                    _copy(o_hbm, scratch, sems, (last - d) % NBUF, last - d,
                          BLOCK_N, k).wait()

    return _body


def kernel(x, y, memory, params):
    del y
    nblocks = pl.cdiv(NROWS, BLOCK_N)
    tail = NROWS - (nblocks - 1) * BLOCK_N
    out_t = pl.pallas_call(
        _make_body(nblocks, tail),
        grid=(nblocks,),
        in_specs=[
            pl.BlockSpec(memory_space=pltpu.SMEM),
            pl.BlockSpec((BATCH, FEAT), lambda j: (0, 0)),
            pl.BlockSpec((BLOCK_N, FEAT), lambda j: (j, 0)),
        ],
        out_specs=pl.BlockSpec(memory_space=pl.ANY),
        out_shape=jax.ShapeDtypeStruct((NROWS, BATCH), jnp.float32),
        scratch_shapes=[
            pltpu.VMEM((NBUF, BLOCK_N, BATCH), jnp.float32),
            pltpu.VMEM((BATCH, FEAT), jnp.bfloat16),
            pltpu.SemaphoreType.DMA((NBUF, SPLIT)),
        ],
        compiler_params=pltpu.CompilerParams(
            dimension_semantics=("arbitrary",),
        ),
    )(params, x, memory)
    return out_t.T


# speedup vs baseline: 1.0386x; 1.0000x over previous
import jax
import jax.numpy as jnp
from jax import lax
from jax.experimental import pallas as pl
from jax.experimental.pallas import tpu as pltpu

BATCH = 4096
FEAT = 128
NROWS = 100000
BLOCK_N = 800
NBUF = 4
SPLIT = 4
COLS_PER = BATCH // SPLIT


def _make_body(nblocks, tail):
    last = nblocks - 1

    def _copy(o_hbm, scratch, sems, s, block, nrows, k):
        band = nrows // SPLIT
        return pltpu.make_async_copy(
            scratch.at[s, pl.ds(k * band, band), :],
            o_hbm.at[pl.ds(block * BLOCK_N + k * band, band), :],
            sems.at[s, k],
        )

    def _body(params_ref, x_ref, m_ref, o_hbm, scratch, xbf, sems):
        i = pl.program_id(0)
        s = lax.rem(i, NBUF)
        inv_t = 1.0 / params_ref[0]

        @pl.when(i == 0)
        def _():
            xbf[...] = x_ref[...].astype(jnp.bfloat16)

        @pl.when(i >= NBUF)
        def _():
            for k in range(SPLIT):
                _copy(o_hbm, scratch, sems, s, i - NBUF, BLOCK_N, k).wait()

        half = BLOCK_N // 2

        @pl.when(i < last)
        def _():
            for h in range(2):
                acc = lax.dot_general(
                    m_ref[pl.ds(h * half, half), :].astype(jnp.bfloat16),
                    xbf[...],
                    dimension_numbers=(((1,), (1,)), ((), ())),
                    preferred_element_type=jnp.float32,
                )
                scratch[s, pl.ds(h * half, half), :] = acc * inv_t
                for k in range(h * (SPLIT // 2), (h + 1) * (SPLIT // 2)):
                    _copy(o_hbm, scratch, sems, s, i, BLOCK_N, k).start()

        @pl.when(i == last)
        def _():
            s_last = last % NBUF
            acc = lax.dot_general(
                m_ref[pl.ds(0, tail), :].astype(jnp.bfloat16), xbf[...],
                dimension_numbers=(((1,), (1,)), ((), ())),
                preferred_element_type=jnp.float32,
            )
            scratch[s_last, pl.ds(0, tail), :] = acc * inv_t
            for k in range(SPLIT):
                _copy(o_hbm, scratch, sems, s_last, last, tail, k).start()
            for k in range(SPLIT):
                _copy(o_hbm, scratch, sems, s_last, last, tail, k).wait()
            for d in range(1, min(NBUF, nblocks)):
                for k in range(SPLIT):
                    _copy(o_hbm, scratch, sems, (last - d) % NBUF, last - d,
                          BLOCK_N, k).wait()

    return _body


def kernel(x, y, memory, params):
    del y
    nblocks = pl.cdiv(NROWS, BLOCK_N)
    tail = NROWS - (nblocks - 1) * BLOCK_N
    out_t = pl.pallas_call(
        _make_body(nblocks, tail),
        grid=(nblocks,),
        in_specs=[
            pl.BlockSpec(memory_space=pltpu.SMEM),
            pl.BlockSpec((BATCH, FEAT), lambda j: (0, 0)),
            pl.BlockSpec((BLOCK_N, FEAT), lambda j: (j, 0)),
        ],
        out_specs=pl.BlockSpec(memory_space=pl.ANY),
        out_shape=jax.ShapeDtypeStruct((NROWS, BATCH), jnp.float32),
        scratch_shapes=[
            pltpu.VMEM((NBUF, BLOCK_N, BATCH), jnp.float32),
            pltpu.VMEM((BATCH, FEAT), jnp.bfloat16),
            pltpu.SemaphoreType.DMA((NBUF, SPLIT)),
        ],
        compiler_params=pltpu.CompilerParams(
            dimension_semantics=("arbitrary",),
        ),
    )(params, x, memory)
    return out_t.T
